# Initial kernel scaffold; baseline (speedup 1.0000x reference)
import jax
import jax.numpy as jnp
from jax.experimental import pallas as pl


def kernel(x, edge_index, W1, att1, fc1W, fc1b, W2, att2, fc2W, fc2b):
    raise NotImplementedError("write your pallas kernel here")


# speedup vs baseline: 1.0392x; 1.0392x over previous
import functools
import jax
import jax.numpy as jnp
from jax import lax
from jax.experimental import pallas as pl
from jax.experimental.pallas import tpu as pltpu

N = 10000
E = 80000
D_IN = 512
H = 6
O1 = 16
O2 = 223
O2P = 224

BN = 1000


def _k1_body(x_ref, w_ref, a_ref, feat_ref, atab_ref):
    xt = jnp.dot(x_ref[:], w_ref[:], preferred_element_type=jnp.float32)
    feat_ref[:] = xt
    atab_ref[:] = jnp.dot(xt, a_ref[:], preferred_element_type=jnp.float32)


def _tc_k1(x, w1, ablk1):
    return pl.pallas_call(
        _k1_body,
        grid=(N // BN,),
        in_specs=[
            pl.BlockSpec((BN, D_IN), lambda i: (i, 0)),
            pl.BlockSpec((D_IN, H * O1), lambda i: (0, 0)),
            pl.BlockSpec((H * O1, 16), lambda i: (0, 0)),
        ],
        out_specs=[
            pl.BlockSpec((BN, H * O1), lambda i: (i, 0)),
            pl.BlockSpec((BN, 16), lambda i: (i, 0)),
        ],
        out_shape=[
            jax.ShapeDtypeStruct((N, H * O1), jnp.float32),
            jax.ShapeDtypeStruct((N, 16), jnp.float32),
        ],
    )(x, w1, ablk1)


def _k4_body(o1_ref, fw_ref, fb_ref, w2_ref, a_ref, feat_ref, atab_ref):
    h = jnp.dot(o1_ref[:], fw_ref[:], preferred_element_type=jnp.float32)
    h = jnp.maximum(h + fb_ref[:], 0.0)
    f2 = jnp.dot(h, w2_ref[:], preferred_element_type=jnp.float32)
    feat_ref[:] = f2
    atab_ref[:] = jnp.dot(f2, a_ref[:], preferred_element_type=jnp.float32)


def _tc_k4(out1, fc1wt, fc1b, w2p, ablk2):
    return pl.pallas_call(
        _k4_body,
        grid=(N // BN,),
        in_specs=[
            pl.BlockSpec((BN, H * O1), lambda i: (i, 0)),
            pl.BlockSpec((H * O1, O1), lambda i: (0, 0)),
            pl.BlockSpec((1, O1), lambda i: (0, 0)),
            pl.BlockSpec((O1, H * O2P), lambda i: (0, 0)),
            pl.BlockSpec((H * O2P, 16), lambda i: (0, 0)),
        ],
        out_specs=[
            pl.BlockSpec((BN, H * O2P), lambda i: (i, 0)),
            pl.BlockSpec((BN, 16), lambda i: (i, 0)),
        ],
        out_shape=[
            jax.ShapeDtypeStruct((N, H * O2P), jnp.float32),
            jax.ShapeDtypeStruct((N, 16), jnp.float32),
        ],
    )(out1, fc1wt, fc1b.reshape(1, O1), w2p, ablk2)


def _k5_body(o2_ref, fw_ref, fb_ref, out_ref):
    z = jnp.dot(o2_ref[:], fw_ref[:], preferred_element_type=jnp.float32)
    z = z + fb_ref[:]
    m = jnp.max(z, axis=1, keepdims=True)
    ez = jnp.exp(z - m)
    out_ref[:] = ez / jnp.sum(ez, axis=1, keepdims=True)


def _tc_k5(out2, fc2wt, fc2b):
    return pl.pallas_call(
        _k5_body,
        grid=(N // BN,),
        in_specs=[
            pl.BlockSpec((BN, H * O2P), lambda i: (i, 0)),
            pl.BlockSpec((H * O2P, O2), lambda i: (0, 0)),
            pl.BlockSpec((1, O2), lambda i: (0, 0)),
        ],
        out_specs=pl.BlockSpec((BN, O2), lambda i: (i, 0)),
        out_shape=jax.ShapeDtypeStruct((N, O2), jnp.float32),
    )(out2, fc2wt, fc2b.reshape(1, O2))


def _seg_softmax(alpha, idx, n):
    amax = jax.ops.segment_max(alpha, idx, num_segments=n)
    amax = jnp.where(jnp.isfinite(amax), amax, 0.0)
    e = jnp.exp(alpha - amax[idx])
    s = jax.ops.segment_sum(e, idx, num_segments=n)
    return e / (s[idx] + 1e-16)


def _edge_stage(atab, he0, he1):
    ai = atab[:, :H]
    aj = atab[:, 8:8 + H]
    alpha = ai[he0] + aj[he1]
    alpha = jnp.where(alpha > 0, alpha, 0.2 * alpha)
    alpha = _seg_softmax(alpha, he0, N)
    deg = jax.ops.segment_sum(jnp.ones((E,), jnp.float32), he0, num_segments=N)
    dn = jnp.where(deg > 0, 1.0 / deg, 0.0)
    w1 = dn[he1][:, None] * alpha
    w2 = dn[he0][:, None] * alpha
    return w1, w2


def _pass(feat, w, src, dst, ocp):
    m = w[:, :, None] * feat.reshape(N, H, ocp)[src]
    return jax.ops.segment_sum(m, dst, num_segments=N).reshape(N, H * ocp)


def _prep(W1, att1, W2, att2, fc1W, fc2W):
    attI1 = att1[0, :, :O1]
    attJ1 = att1[0, :, O1:]
    ablk1 = jnp.zeros((H * O1, 16), jnp.float32)
    hh = jnp.arange(H).repeat(O1)
    oo = jnp.tile(jnp.arange(O1), H)
    ablk1 = ablk1.at[jnp.arange(H * O1), hh].set(attI1[hh, oo])
    ablk1 = ablk1.at[jnp.arange(H * O1), 8 + hh].set(attJ1[hh, oo])

    w2p = jnp.pad(W2.reshape(O1, H, O2), ((0, 0), (0, 0), (0, O2P - O2))
                  ).reshape(O1, H * O2P)
    attI2 = att2[0, :, :O2]
    attJ2 = att2[0, :, O2:]
    ablk2 = jnp.zeros((H * O2P, 16), jnp.float32)
    h2 = jnp.arange(H).repeat(O2P)
    o2 = jnp.tile(jnp.arange(O2P), H)
    val_i = jnp.where(o2 < O2, attI2[h2, jnp.minimum(o2, O2 - 1)], 0.0)
    val_j = jnp.where(o2 < O2, attJ2[h2, jnp.minimum(o2, O2 - 1)], 0.0)
    ablk2 = ablk2.at[jnp.arange(H * O2P), h2].set(val_i)
    ablk2 = ablk2.at[jnp.arange(H * O2P), 8 + h2].set(val_j)

    fc1wt = fc1W.T
    fc2wt = jnp.pad(fc2W.T.reshape(H, O2, O2), ((0, 0), (0, O2P - O2), (0, 0))
                    ).reshape(H * O2P, O2)
    return ablk1, w2p, ablk2, fc1wt, fc2wt


def kernel(x, edge_index, W1, att1, fc1W, fc1b, W2, att2, fc2W, fc2b):
    ablk1, w2p, ablk2, fc1wt, fc2wt = _prep(W1, att1, W2, att2, fc1W, fc2W)
    he0 = edge_index[0]
    he1 = edge_index[1]

    feat1, atab1 = _tc_k1(x, W1, ablk1)
    w1a, w1b = _edge_stage(atab1, he0, he1)
    mid1 = _pass(feat1, w1a, he0, he1, O1)
    out1 = _pass(mid1, w1b, he1, he0, O1)

    feat2, atab2 = _tc_k4(out1, fc1wt, fc1b, w2p, ablk2)
    w2a, w2b = _edge_stage(atab2, he0, he1)
    mid2 = _pass(feat2, w2a, he0, he1, O2P)
    out2 = _pass(mid2, w2b, he1, he0, O2P)

    return _tc_k5(out2, fc2wt, fc2b)
